# baseline (speedup 1.0000x reference)
import functools

import jax
import jax.numpy as jnp
from jax.experimental import pallas as pl
from jax.experimental.pallas import tpu as pltpu

_BM = 2048
_NBUF = 5
_NROW = 4


def _mm_kernel(x_ref, f_ref, o_hbm, o_buf, o_tail, sems, tail_sem, *,
               n_steps, n_full, tail, rb):
    i = pl.program_id(0)
    slot = jax.lax.rem(i, _NBUF)

    def tile_copies(step_start, sl):
        return [
            pltpu.make_async_copy(
                o_buf.at[sl, pl.ds(r * rb, rb)],
                o_hbm.at[pl.ds(r * rb, rb), pl.ds(step_start, _BM)],
                sems.at[sl, r])
            for r in range(_NROW)
        ]

    @pl.when(i >= _NBUF)
    def _wait_slot():
        for c in tile_copies((i - _NBUF) * _BM, slot):
            c.wait()

    result = jax.lax.dot_general(
        x_ref[...], f_ref[...],
        dimension_numbers=(((1,), (1,)), ((), ())),
        preferred_element_type=jnp.float32)

    @pl.when(i < n_full)
    def _start_full():
        o_buf[slot] = result
        for c in tile_copies(i * _BM, slot):
            c.start()

    if tail:
        @pl.when(i == n_full)
        def _start_tail():
            o_tail[...] = result[:, :tail]
            pltpu.make_async_copy(
                o_tail,
                o_hbm.at[:, pl.ds(n_full * _BM, tail)],
                tail_sem).start()

    @pl.when(i == n_steps - 1)
    def _drain():
        for step in range(max(n_steps - _NBUF, 0), min(n_full, n_steps)):
            for c in tile_copies(step * _BM, step % _NBUF):
                c.wait()
        if tail:
            pltpu.make_async_copy(
                o_tail,
                o_hbm.at[:, pl.ds(n_full * _BM, tail)],
                tail_sem).wait()


def kernel(inputs, indexes, IoU, update_flag, features):
    B, D = inputs.shape
    M = features.shape[0]
    n_steps = pl.cdiv(M, _BM)
    n_full = M // _BM
    tail = M - n_full * _BM
    return pl.pallas_call(
        functools.partial(_mm_kernel, n_steps=n_steps, n_full=n_full,
                          tail=tail, rb=B // _NROW),
        grid=(n_steps,),
        in_specs=[
            pl.BlockSpec((B, D), lambda i: (0, 0)),
            pl.BlockSpec((_BM, D), lambda i: (i, 0)),
        ],
        out_specs=pl.BlockSpec(memory_space=pl.ANY),
        out_shape=jax.ShapeDtypeStruct((B, M), jnp.float32),
        scratch_shapes=[
            pltpu.VMEM((_NBUF, B, _BM), jnp.float32),
            pltpu.VMEM((B, tail if tail else 128), jnp.float32),
            pltpu.SemaphoreType.DMA((_NBUF, _NROW)),
            pltpu.SemaphoreType.DMA,
        ],
    )(inputs, features)


# speedup vs baseline: 3.4766x; 3.4766x over previous
import jax
import jax.numpy as jnp
from jax.experimental import pallas as pl

_BM = 2048


def _mm_kernel(f_ref, x_ref, o_ref):
    o_ref[...] = jax.lax.dot_general(
        f_ref[...], x_ref[...],
        dimension_numbers=(((1,), (1,)), ((), ())),
        preferred_element_type=jnp.float32)


def kernel(inputs, indexes, IoU, update_flag, features):
    B, D = inputs.shape
    M = features.shape[0]
    ot = pl.pallas_call(
        _mm_kernel,
        grid=(pl.cdiv(M, _BM),),
        in_specs=[
            pl.BlockSpec((_BM, D), lambda i: (i, 0)),
            pl.BlockSpec((B, D), lambda i: (0, 0)),
        ],
        out_specs=pl.BlockSpec((_BM, B), lambda i: (i, 0)),
        out_shape=jax.ShapeDtypeStruct((M, B), jnp.float32),
    )(features, inputs)
    return ot.T


# speedup vs baseline: 3.5481x; 1.0206x over previous
import jax
import jax.numpy as jnp
from jax.experimental import pallas as pl

_BM = 4096


def _mm_kernel(f_ref, x_ref, o_ref):
    o_ref[...] = jax.lax.dot_general(
        f_ref[...], x_ref[...],
        dimension_numbers=(((1,), (1,)), ((), ())),
        preferred_element_type=jnp.float32)


def kernel(inputs, indexes, IoU, update_flag, features):
    B, D = inputs.shape
    M = features.shape[0]
    ot = pl.pallas_call(
        _mm_kernel,
        grid=(pl.cdiv(M, _BM),),
        in_specs=[
            pl.BlockSpec((_BM, D), lambda i: (i, 0)),
            pl.BlockSpec((B, D), lambda i: (0, 0)),
        ],
        out_specs=pl.BlockSpec((_BM, B), lambda i: (i, 0)),
        out_shape=jax.ShapeDtypeStruct((M, B), jnp.float32),
    )(features, inputs)
    return ot.T
